# baseline (speedup 1.0000x reference)
import jax
import jax.numpy as jnp
from jax import lax
from jax.experimental import pallas as pl
from jax.experimental.pallas import tpu as pltpu
from jax.experimental.pallas import tpu_sc as plsc

_BATCH = 16384
_PERIOD = 50
_HID = 16

_NC = 2
_NS = 16
_NW = _NC * _NS
_ROWS = _BATCH // _NW


def _body(w_hbm, out_hbm, w_v, sem):
    wid = lax.axis_index("s") * _NC + lax.axis_index("c")
    base = wid * _ROWS
    pltpu.sync_copy(w_hbm.at[pl.ds(base, _ROWS)], w_v)
    copies = [
        pltpu.async_copy(w_v, out_hbm.at[pl.ds(base, _ROWS), p, :], sem)
        for p in range(_PERIOD)
    ]
    for c in copies:
        c.wait()


def kernel(x, W):
    mesh = plsc.VectorSubcoreMesh(core_axis_name="c", subcore_axis_name="s")
    k = pl.kernel(
        _body,
        out_type=jax.ShapeDtypeStruct((_BATCH, _PERIOD, _HID), jnp.float32),
        mesh=mesh,
        scratch_types=[
            pltpu.VMEM((_ROWS, _HID), jnp.float32),
            pltpu.SemaphoreType.DMA,
        ],
    )
    return k(W)
